# baseline (speedup 1.0000x reference)
import functools

import jax
import jax.numpy as jnp
from jax import lax
from jax.experimental import pallas as pl
from jax.experimental.pallas import tpu as pltpu
from jax.experimental.pallas import tpu_sc as plsc

N = 10000
E = 320000
H = 128
B = 100
EPS = 1e-5

NC = 2
NS = 16
NW = NC * NS

f32 = jnp.float32
i32 = jnp.int32


def _proa_body(x_ref, naW_ref, nab_ref, eW_ref, eb_ref,
               na0_ref, na1_ref, a_ref, b_ref):
    x = x_ref[...]
    l = jnp.dot(x, naW_ref[...], preferred_element_type=f32) + nab_ref[...]
    dn = l[:, 0:1] - l[:, 1:2]
    na0 = 1.0 / (1.0 + jnp.exp(-dn))
    na0_ref[...] = na0
    na1_ref[...] = 1.0 - na0

    eW = eW_ref[...]
    P = jnp.dot(x, eW[:H, :], preferred_element_type=f32)
    Q = jnp.dot(x, eW[H:, :], preferred_element_type=f32)
    eb = eb_ref[...]
    a_ref[...] = (P[:, 0:1] - P[:, 1:2]) + (eb[0:1, 0:1] - eb[0:1, 1:2])
    b_ref[...] = Q[:, 0:1] - Q[:, 1:2]


_pro_a = pl.pallas_call(
    _proa_body,
    out_shape=[
        jax.ShapeDtypeStruct((N, 1), f32),
        jax.ShapeDtypeStruct((N, 1), f32),
        jax.ShapeDtypeStruct((N, 1), f32),
        jax.ShapeDtypeStruct((N, 1), f32),
    ],
)


def _prob_body(x_ref, na0_ref, na1_ref, gc_ref, bc_ref,
               go_ref, bo_ref, Wc_ref, Wo_ref, h2_ref):
    x = x_ref[...]
    for j, (na_r, g_r, be_r, W_r) in enumerate(
            ((na0_ref, gc_ref, bc_ref, Wc_ref),
             (na1_ref, go_ref, bo_ref, Wo_ref))):
        xm = na_r[...] * x
        m = jnp.mean(xm, axis=0, keepdims=True)
        var = jnp.mean((xm - m) ** 2, axis=0, keepdims=True)
        bnv = (xm - m) / jnp.sqrt(var + EPS) * g_r[...] + be_r[...]
        h2_ref[j] = jnp.dot(bnv, W_r[...], preferred_element_type=f32)


_pro_b = pl.pallas_call(
    _prob_body,
    out_shape=jax.ShapeDtypeStruct((2, N, H), f32),
)


_sc_mesh = plsc.VectorSubcoreMesh(core_axis_name="c", subcore_axis_name="s")

EW1 = E // NW
EC1 = 2000
NCH1 = EW1 // EC1


@functools.partial(
    pl.kernel,
    out_type=[
        jax.ShapeDtypeStruct((E,), f32),
        jax.ShapeDtypeStruct((NW, 1, N), f32),
        jax.ShapeDtypeStruct((NW, 1, N), f32),
    ],
    mesh=_sc_mesh,
    compiler_params=pltpu.CompilerParams(needs_layout_passes=False),
    scratch_types=[
        pltpu.VMEM((N,), f32),
        pltpu.VMEM((N,), f32),
        pltpu.VMEM((N,), f32),
        pltpu.VMEM((N,), f32),
        pltpu.VMEM((EC1,), i32),
        pltpu.VMEM((EC1,), i32),
        pltpu.VMEM((EC1,), f32),
    ],
)
def _edge_pass1(row_hbm, col_hbm, a_hbm, b_hbm,
                ew_hbm, dpc_hbm, dpo_hbm,
                a_v, b_v, dc_v, do_v, row_v, col_v, ew_v):
    cid = lax.axis_index("c")
    sid = lax.axis_index("s")
    wid = sid * NC + cid
    pltpu.sync_copy(a_hbm, a_v)
    pltpu.sync_copy(b_hbm, b_v)

    def zero16(i, _):
        z = jnp.zeros((16,), f32)
        dc_v[pl.ds(i * 16, 16)] = z
        do_v[pl.ds(i * 16, 16)] = z
        return 0
    lax.fori_loop(0, N // 16, zero16, 0)

    base0 = wid * EW1
    for k in range(NCH1):
        base = base0 + k * EC1
        pltpu.sync_copy(row_hbm.at[pl.ds(base, EC1)], row_v)
        pltpu.sync_copy(col_hbm.at[pl.ds(base, EC1)], col_v)

        def body(i, _):
            r = row_v[pl.ds(i * 16, 16)]
            c = col_v[pl.ds(i * 16, 16)]
            av = plsc.load_gather(a_v, [r])
            bv = plsc.load_gather(b_v, [c])
            ew = 1.0 / (1.0 + jnp.exp(-(av + bv)))
            ew_v[pl.ds(i * 16, 16)] = ew
            plsc.addupdate_scatter(dc_v, [r], ew)
            plsc.addupdate_scatter(do_v, [r], 1.0 - ew)
            return 0
        lax.fori_loop(0, EC1 // 16, body, 0)
        pltpu.sync_copy(ew_v, ew_hbm.at[pl.ds(base, EC1)])

    pltpu.sync_copy(dc_v, dpc_hbm.at[wid, 0])
    pltpu.sync_copy(do_v, dpo_hbm.at[wid, 0])


def _dinv_body(dpc_ref, dpo_ref, d2_ref):
    for j, dp in enumerate((dpc_ref, dpo_ref)):
        deg = jnp.sum(dp[...], axis=0) + 1.0
        d2_ref[j] = 1.0 / jnp.sqrt(deg)


_dinv = pl.pallas_call(
    _dinv_body,
    out_shape=jax.ShapeDtypeStruct((2, 1, N), f32),
)


EC2 = 128
SLAB = 4
NSL = E // (EC2 * SLAB)
SPS = NSL // NS
REMS = NSL - SPS * NS
NP2 = 10240
NPS = NP2 // NS


bf16 = jnp.bfloat16


@functools.partial(
    pl.kernel,
    out_type=jax.ShapeDtypeStruct((2, NP2, H), f32),
    mesh=_sc_mesh,
    compiler_params=pltpu.CompilerParams(needs_layout_passes=False),
    scratch_types=[
        pltpu.VMEM((N,), f32),
        pltpu.VMEM((2, 2 * SLAB, 128), i32),
        pltpu.VMEM((2, SLAB, 128), f32),
        pltpu.VMEM((SLAB, 128), f32),
        pltpu.VMEM((SLAB, 128), i32),
        pltpu.VMEM((2, 128), i32),
        pltpu.VMEM((2 * EC2, H), f32),
        pltpu.VMEM_SHARED((NP2, H), f32),
        pltpu.SemaphoreType.DMA,
        pltpu.SemaphoreType.DMA,
        pltpu.SemaphoreType.DMA,
        pltpu.SemaphoreType.DMA,
        pltpu.SemaphoreType.DMA,
    ],
)
def _edge_pass2(rc_hbm, ew3_hbm, dinv2_hbm, hflat_hbm,
                out_hbm,
                dinv_v, rcs_v, ews_v, norm_v, gidx_v, csc_v,
                rows_v, acc, sg0, sg1, ss0, ss1, spf):
    cid = lax.axis_index("c")
    sid = lax.axis_index("s")
    cidf = cid.astype(f32)
    sgn = 1.0 - 2.0 * cidf
    cidN = cid * N
    sg = (sg0, sg1)
    ss = (ss0, ss1)

    pltpu.sync_copy(dinv2_hbm.at[cid, 0], dinv_v)

    def zb_body(i, _):
        rows_v[i // 8, pl.ds((i % 8) * 16, 16)] = jnp.zeros((16,), f32)
        return 0
    lax.fori_loop(0, EC2 * 8, zb_body, 0)
    for j in range(NPS // EC2):
        pltpu.sync_copy(rows_v.at[pl.ds(0, EC2)],
                        acc.at[pl.ds(sid * NPS + j * EC2, EC2)])
    plsc.subcore_barrier()

    def pf_start(slab_k, p):
        pltpu.async_copy(rc_hbm.at[slab_k], rcs_v.at[p], spf)
        pltpu.async_copy(ew3_hbm.at[slab_k], ews_v.at[p], spf)

    def pf_wait(p):
        pltpu.make_async_copy(rc_hbm.at[0], rcs_v.at[p], spf).wait()
        pltpu.make_async_copy(ew3_hbm.at[0], ews_v.at[p], spf).wait()

    def gather_start(g, b):
        pltpu.async_copy(hflat_hbm.at[gidx_v.at[g]],
                         rows_v.at[pl.ds(b * EC2, EC2)], sg[b])

    def gather_wait(b):
        pltpu.make_async_copy(hflat_hbm.at[gidx_v.at[0]],
                              rows_v.at[pl.ds(b * EC2, EC2)], sg[b]).wait()

    def scat_start(b):
        pltpu.async_copy(rows_v.at[pl.ds(b * EC2, EC2)],
                         acc.at[csc_v.at[b]], ss[b], add=True)

    def scat_wait(b):
        pltpu.make_async_copy(rows_v.at[pl.ds(b * EC2, EC2)],
                              acc.at[csc_v.at[b]], ss[b]).wait()

    def slab_fn(p, nonfirst):
        for g in range(SLAB):
            def nbody(i, _):
                sl = pl.ds(i * 16, 16)
                r = rcs_v[p, g, sl]
                c = rcs_v[p, SLAB + g, sl]
                w = cidf + sgn * ews_v[p, g, sl]
                nr = plsc.load_gather(dinv_v, [r])
                ncv = plsc.load_gather(dinv_v, [c])
                norm_v[g, sl] = nr * w * ncv
                gidx_v[g, sl] = r + cidN
                return 0
            lax.fori_loop(0, 8, nbody, 0, unroll=2)
        @pl.when(nonfirst)
        def _():
            scat_wait(0)
        gather_start(0, 0)
        for c in range(SLAB):
            b = c % 2
            if c < SLAB - 1:
                if c == 0:
                    @pl.when(nonfirst)
                    def _():
                        scat_wait(1 - b)
                else:
                    scat_wait(1 - b)
                gather_start(c + 1, 1 - b)
            gather_wait(b)

            def sbody(i, _):
                nv = norm_v[c, pl.ds(i * 16, 16)]
                csc_v[b, pl.ds(i * 16, 16)] = rcs_v[p, SLAB + c,
                                                    pl.ds(i * 16, 16)]
                for lane in range(16):
                    s = nv[lane]
                    r0 = b * EC2 + i * 16 + lane
                    for h8 in range(8):
                        sl = pl.ds(h8 * 16, 16)
                        rows_v[r0, sl] = rows_v[r0, sl] * s
                return 0
            lax.fori_loop(0, 8, sbody, 0, unroll=2)
            scat_start(b)

    pf_start(sid * SPS, 0)

    def slab_body(j, _):
        p = j % 2
        pf_wait(p)

        @pl.when(j + 1 < SPS)
        def _():
            pf_start(sid * SPS + j + 1, 1 - p)
        slab_fn(p, j > 0)
        return 0
    lax.fori_loop(0, SPS, slab_body, 0)

    @pl.when(sid < REMS)
    def _():
        p = SPS % 2
        pf_start(NS * SPS + sid, p)
        pf_wait(p)
        slab_fn(p, jnp.bool_(True))

    scat_wait(0)
    scat_wait(1)

    plsc.subcore_barrier()
    pltpu.sync_copy(acc.at[pl.ds(sid * NPS, NPS)],
                    out_hbm.at[cid, pl.ds(sid * NPS, NPS)])


GB = 2000


def _fin_body(out2_ref, h2_ref, dsq_ref, bc_ref, bo_ref,
              mo_ref, mc_ref, mco_ref):
    o = out2_ref[...].astype(f32)
    hh = h2_ref[...]
    di = dsq_ref[...]
    oc = o[0] + di[0] * di[0] * hh[0] + bc_ref[...]
    oo = o[1] + di[1] * di[1] * hh[1] + bo_ref[...]
    mc = jnp.mean(jax.nn.relu(oc), axis=-1, keepdims=True)
    mo = jnp.mean(jax.nn.relu(oo), axis=-1, keepdims=True)
    mo_ref[...] = mo
    mc_ref[...] = mc
    mco_ref[...] = mc + mo


_finale = pl.pallas_call(
    _fin_body,
    grid=(N // GB,),
    in_specs=[
        pl.BlockSpec((2, GB, H), lambda i: (0, i, 0)),
        pl.BlockSpec((2, GB, H), lambda i: (0, i, 0)),
        pl.BlockSpec((2, GB, 1), lambda i: (0, i, 0)),
        pl.BlockSpec((1, H), lambda i: (0, 0)),
        pl.BlockSpec((1, H), lambda i: (0, 0)),
    ],
    out_specs=[
        pl.BlockSpec((GB, 1), lambda i: (i, 0)),
        pl.BlockSpec((GB, 1), lambda i: (i, 0)),
        pl.BlockSpec((GB, 1), lambda i: (i, 0)),
    ],
    out_shape=[
        jax.ShapeDtypeStruct((N, 1), f32),
        jax.ShapeDtypeStruct((N, 1), f32),
        jax.ShapeDtypeStruct((N, 1), f32),
    ],
)


def kernel(x, edge_index, batch, edge_att_W, edge_att_b, node_att_W,
           node_att_b, bnc_gamma, bnc_beta, bno_gamma, bno_beta,
           conv_c_W, conv_c_b, conv_o_W, conv_o_b):
    row = edge_index[0]
    col = edge_index[1]
    rc = jnp.concatenate(
        [row.reshape(NSL, SLAB, 128), col.reshape(NSL, SLAB, 128)], axis=1)
    na0, na1, a1, b1 = _pro_a(
        x, node_att_W, node_att_b.reshape(1, 2), edge_att_W,
        edge_att_b.reshape(1, 2))
    h2 = _pro_b(
        x, na0, na1, bnc_gamma.reshape(1, H), bnc_beta.reshape(1, H),
        bno_gamma.reshape(1, H), bno_beta.reshape(1, H), conv_c_W, conv_o_W)
    ew, dpc, dpo = _edge_pass1(row, col, a1.reshape(-1), b1.reshape(-1))
    dinv2 = _dinv(dpc, dpo)
    out2 = _edge_pass2(
        rc, ew.reshape(NSL, SLAB, 128), dinv2, h2.reshape(2 * N, H))
    mo, mc, mco = _finale(out2, h2, dinv2.reshape(2, N, 1),
                          conv_c_b.reshape(1, H), conv_o_b.reshape(1, H))
    return (mo.reshape(B, B), mc.reshape(B, B), mco.reshape(B, B),
            na1.reshape(-1))


# speedup vs baseline: 1.0316x; 1.0316x over previous
import functools

import jax
import jax.numpy as jnp
from jax import lax
from jax.experimental import pallas as pl
from jax.experimental.pallas import tpu as pltpu
from jax.experimental.pallas import tpu_sc as plsc

N = 10000
E = 320000
H = 128
B = 100
EPS = 1e-5

NC = 2
NS = 16
NW = NC * NS

f32 = jnp.float32
i32 = jnp.int32


def _proa_body(x_ref, naW_ref, nab_ref, eW_ref, eb_ref,
               na0_ref, na1_ref, a_ref, b_ref):
    x = x_ref[...]
    l = jnp.dot(x, naW_ref[...], preferred_element_type=f32) + nab_ref[...]
    dn = l[:, 0:1] - l[:, 1:2]
    na0 = 1.0 / (1.0 + jnp.exp(-dn))
    na0_ref[...] = na0
    na1_ref[...] = 1.0 - na0

    eW = eW_ref[...]
    P = jnp.dot(x, eW[:H, :], preferred_element_type=f32)
    Q = jnp.dot(x, eW[H:, :], preferred_element_type=f32)
    eb = eb_ref[...]
    a_ref[...] = (P[:, 0:1] - P[:, 1:2]) + (eb[0:1, 0:1] - eb[0:1, 1:2])
    b_ref[...] = Q[:, 0:1] - Q[:, 1:2]


_pro_a = pl.pallas_call(
    _proa_body,
    out_shape=[
        jax.ShapeDtypeStruct((N, 1), f32),
        jax.ShapeDtypeStruct((N, 1), f32),
        jax.ShapeDtypeStruct((N, 1), f32),
        jax.ShapeDtypeStruct((N, 1), f32),
    ],
)


def _prob_body(x_ref, na0_ref, na1_ref, gc_ref, bc_ref,
               go_ref, bo_ref, Wc_ref, Wo_ref, h2_ref):
    x = x_ref[...]
    for j, (na_r, g_r, be_r, W_r) in enumerate(
            ((na0_ref, gc_ref, bc_ref, Wc_ref),
             (na1_ref, go_ref, bo_ref, Wo_ref))):
        xm = na_r[...] * x
        m = jnp.mean(xm, axis=0, keepdims=True)
        var = jnp.mean((xm - m) ** 2, axis=0, keepdims=True)
        bnv = (xm - m) / jnp.sqrt(var + EPS) * g_r[...] + be_r[...]
        h2_ref[j] = jnp.dot(bnv, W_r[...], preferred_element_type=f32)


_pro_b = pl.pallas_call(
    _prob_body,
    out_shape=jax.ShapeDtypeStruct((2, N, H), f32),
)


_sc_mesh = plsc.VectorSubcoreMesh(core_axis_name="c", subcore_axis_name="s")

EW1 = E // NW
EC1 = 2000
NCH1 = EW1 // EC1


@functools.partial(
    pl.kernel,
    out_type=[
        jax.ShapeDtypeStruct((E,), f32),
        jax.ShapeDtypeStruct((NW, 1, N), f32),
        jax.ShapeDtypeStruct((NW, 1, N), f32),
    ],
    mesh=_sc_mesh,
    compiler_params=pltpu.CompilerParams(needs_layout_passes=False),
    scratch_types=[
        pltpu.VMEM((N,), f32),
        pltpu.VMEM((N,), f32),
        pltpu.VMEM((N,), f32),
        pltpu.VMEM((N,), f32),
        pltpu.VMEM((EC1,), i32),
        pltpu.VMEM((EC1,), i32),
        pltpu.VMEM((EC1,), f32),
    ],
)
def _edge_pass1(row_hbm, col_hbm, a_hbm, b_hbm,
                ew_hbm, dpc_hbm, dpo_hbm,
                a_v, b_v, dc_v, do_v, row_v, col_v, ew_v):
    cid = lax.axis_index("c")
    sid = lax.axis_index("s")
    wid = sid * NC + cid
    pltpu.sync_copy(a_hbm, a_v)
    pltpu.sync_copy(b_hbm, b_v)

    def zero16(i, _):
        z = jnp.zeros((16,), f32)
        dc_v[pl.ds(i * 16, 16)] = z
        do_v[pl.ds(i * 16, 16)] = z
        return 0
    lax.fori_loop(0, N // 16, zero16, 0)

    base0 = wid * EW1
    for k in range(NCH1):
        base = base0 + k * EC1
        pltpu.sync_copy(row_hbm.at[pl.ds(base, EC1)], row_v)
        pltpu.sync_copy(col_hbm.at[pl.ds(base, EC1)], col_v)

        def body(i, _):
            r = row_v[pl.ds(i * 16, 16)]
            c = col_v[pl.ds(i * 16, 16)]
            av = plsc.load_gather(a_v, [r])
            bv = plsc.load_gather(b_v, [c])
            ew = 1.0 / (1.0 + jnp.exp(-(av + bv)))
            ew_v[pl.ds(i * 16, 16)] = ew
            plsc.addupdate_scatter(dc_v, [r], ew)
            plsc.addupdate_scatter(do_v, [r], 1.0 - ew)
            return 0
        lax.fori_loop(0, EC1 // 16, body, 0)
        pltpu.sync_copy(ew_v, ew_hbm.at[pl.ds(base, EC1)])

    pltpu.sync_copy(dc_v, dpc_hbm.at[wid, 0])
    pltpu.sync_copy(do_v, dpo_hbm.at[wid, 0])


def _dinv_body(dpc_ref, dpo_ref, d2_ref):
    for j, dp in enumerate((dpc_ref, dpo_ref)):
        deg = jnp.sum(dp[...], axis=0) + 1.0
        d2_ref[j] = 1.0 / jnp.sqrt(deg)


_dinv = pl.pallas_call(
    _dinv_body,
    out_shape=jax.ShapeDtypeStruct((2, 1, N), f32),
)


EC2 = 128
SLAB = 4
NSL = E // (EC2 * SLAB)
SPS = NSL // NS
REMS = NSL - SPS * NS
NP2 = 10240
NPS = NP2 // NS


bf16 = jnp.bfloat16


@functools.partial(
    pl.kernel,
    out_type=jax.ShapeDtypeStruct((2, NP2, H), f32),
    mesh=_sc_mesh,
    compiler_params=pltpu.CompilerParams(needs_layout_passes=False),
    scratch_types=[
        pltpu.VMEM((N,), f32),
        pltpu.VMEM((2, 2 * SLAB, 128), i32),
        pltpu.VMEM((2, SLAB, 128), f32),
        pltpu.VMEM((SLAB, 128), f32),
        pltpu.VMEM((SLAB, 128), i32),
        pltpu.VMEM((2, 128), i32),
        pltpu.VMEM((2 * EC2, H), f32),
        pltpu.VMEM_SHARED((NP2, H), f32),
        pltpu.SemaphoreType.DMA,
        pltpu.SemaphoreType.DMA,
        pltpu.SemaphoreType.DMA,
        pltpu.SemaphoreType.DMA,
        pltpu.SemaphoreType.DMA,
    ],
)
def _edge_pass2(rc_hbm, ew3_hbm, dinv2_hbm, hflat_hbm,
                out_hbm,
                dinv_v, rcs_v, ews_v, norm_v, gidx_v, csc_v,
                rows_v, acc, sg0, sg1, ss0, ss1, spf):
    cid = lax.axis_index("c")
    sid = lax.axis_index("s")
    cidf = cid.astype(f32)
    sgn = 1.0 - 2.0 * cidf
    cidN = cid * N
    sg = (sg0, sg1)
    ss = (ss0, ss1)

    pltpu.sync_copy(dinv2_hbm.at[cid, 0], dinv_v)

    def zb_body(i, _):
        rows_v[i // 8, pl.ds((i % 8) * 16, 16)] = jnp.zeros((16,), f32)
        return 0
    lax.fori_loop(0, EC2 * 8, zb_body, 0)
    for j in range(NPS // EC2):
        pltpu.sync_copy(rows_v.at[pl.ds(0, EC2)],
                        acc.at[pl.ds(sid * NPS + j * EC2, EC2)])
    plsc.subcore_barrier()

    def pf_start(slab_k, p):
        pltpu.async_copy(rc_hbm.at[slab_k], rcs_v.at[p], spf)
        pltpu.async_copy(ew3_hbm.at[slab_k], ews_v.at[p], spf)

    def pf_wait(p):
        pltpu.make_async_copy(rc_hbm.at[0], rcs_v.at[p], spf).wait()
        pltpu.make_async_copy(ew3_hbm.at[0], ews_v.at[p], spf).wait()

    def gather_start(g, b):
        pltpu.async_copy(hflat_hbm.at[gidx_v.at[g]],
                         rows_v.at[pl.ds(b * EC2, EC2)], sg[b])

    def gather_wait(b):
        pltpu.make_async_copy(hflat_hbm.at[gidx_v.at[0]],
                              rows_v.at[pl.ds(b * EC2, EC2)], sg[b]).wait()

    def scat_start(b):
        pltpu.async_copy(rows_v.at[pl.ds(b * EC2, EC2)],
                         acc.at[csc_v.at[b]], ss[b], add=True)

    def scat_wait(b):
        pltpu.make_async_copy(rows_v.at[pl.ds(b * EC2, EC2)],
                              acc.at[csc_v.at[b]], ss[b]).wait()

    def slab_fn(p, nonfirst):
        for g in range(SLAB):
            def nbody(i, _):
                sl = pl.ds(i * 16, 16)
                r = rcs_v[p, g, sl]
                c = rcs_v[p, SLAB + g, sl]
                w = cidf + sgn * ews_v[p, g, sl]
                nr = plsc.load_gather(dinv_v, [r])
                ncv = plsc.load_gather(dinv_v, [c])
                norm_v[g, sl] = nr * w * ncv
                gidx_v[g, sl] = r + cidN
                return 0
            lax.fori_loop(0, 8, nbody, 0)
        @pl.when(nonfirst)
        def _():
            scat_wait(0)
        gather_start(0, 0)
        for c in range(SLAB):
            b = c % 2
            if c < SLAB - 1:
                if c == 0:
                    @pl.when(nonfirst)
                    def _():
                        scat_wait(1 - b)
                else:
                    scat_wait(1 - b)
                gather_start(c + 1, 1 - b)
            gather_wait(b)

            def sbody(i, _):
                nv = norm_v[c, pl.ds(i * 16, 16)]
                csc_v[b, pl.ds(i * 16, 16)] = rcs_v[p, SLAB + c,
                                                    pl.ds(i * 16, 16)]
                for lane in range(16):
                    s = nv[lane]
                    r0 = b * EC2 + i * 16 + lane
                    for h8 in range(8):
                        sl = pl.ds(h8 * 16, 16)
                        rows_v[r0, sl] = rows_v[r0, sl] * s
                return 0
            lax.fori_loop(0, 8, sbody, 0)
            scat_start(b)

    pf_start(sid * SPS, 0)

    def slab_body(j, _):
        p = j % 2
        pf_wait(p)

        @pl.when(j + 1 < SPS)
        def _():
            pf_start(sid * SPS + j + 1, 1 - p)
        slab_fn(p, j > 0)
        return 0
    lax.fori_loop(0, SPS, slab_body, 0)

    @pl.when(sid < REMS)
    def _():
        p = SPS % 2
        pf_start(NS * SPS + sid, p)
        pf_wait(p)
        slab_fn(p, jnp.bool_(True))

    scat_wait(0)
    scat_wait(1)

    plsc.subcore_barrier()
    pltpu.sync_copy(acc.at[pl.ds(sid * NPS, NPS)],
                    out_hbm.at[cid, pl.ds(sid * NPS, NPS)])


GB = 2000


def _fin_body(out2_ref, h2_ref, dsq_ref, bc_ref, bo_ref,
              mo_ref, mc_ref, mco_ref):
    o = out2_ref[...].astype(f32)
    hh = h2_ref[...]
    di = dsq_ref[...]
    oc = o[0] + di[0] * di[0] * hh[0] + bc_ref[...]
    oo = o[1] + di[1] * di[1] * hh[1] + bo_ref[...]
    mc = jnp.mean(jax.nn.relu(oc), axis=-1, keepdims=True)
    mo = jnp.mean(jax.nn.relu(oo), axis=-1, keepdims=True)
    mo_ref[...] = mo
    mc_ref[...] = mc
    mco_ref[...] = mc + mo


_finale = pl.pallas_call(
    _fin_body,
    grid=(N // GB,),
    in_specs=[
        pl.BlockSpec((2, GB, H), lambda i: (0, i, 0)),
        pl.BlockSpec((2, GB, H), lambda i: (0, i, 0)),
        pl.BlockSpec((2, GB, 1), lambda i: (0, i, 0)),
        pl.BlockSpec((1, H), lambda i: (0, 0)),
        pl.BlockSpec((1, H), lambda i: (0, 0)),
    ],
    out_specs=[
        pl.BlockSpec((GB, 1), lambda i: (i, 0)),
        pl.BlockSpec((GB, 1), lambda i: (i, 0)),
        pl.BlockSpec((GB, 1), lambda i: (i, 0)),
    ],
    out_shape=[
        jax.ShapeDtypeStruct((N, 1), f32),
        jax.ShapeDtypeStruct((N, 1), f32),
        jax.ShapeDtypeStruct((N, 1), f32),
    ],
)


def kernel(x, edge_index, batch, edge_att_W, edge_att_b, node_att_W,
           node_att_b, bnc_gamma, bnc_beta, bno_gamma, bno_beta,
           conv_c_W, conv_c_b, conv_o_W, conv_o_b):
    row = edge_index[0]
    col = edge_index[1]
    rc = jnp.concatenate(
        [row.reshape(NSL, SLAB, 128), col.reshape(NSL, SLAB, 128)], axis=1)
    na0, na1, a1, b1 = _pro_a(
        x, node_att_W, node_att_b.reshape(1, 2), edge_att_W,
        edge_att_b.reshape(1, 2))
    h2 = _pro_b(
        x, na0, na1, bnc_gamma.reshape(1, H), bnc_beta.reshape(1, H),
        bno_gamma.reshape(1, H), bno_beta.reshape(1, H), conv_c_W, conv_o_W)
    ew, dpc, dpo = _edge_pass1(row, col, a1.reshape(-1), b1.reshape(-1))
    dinv2 = _dinv(dpc, dpo)
    out2 = _edge_pass2(
        rc, ew.reshape(NSL, SLAB, 128), dinv2, h2.reshape(2 * N, H))
    mo, mc, mco = _finale(out2, h2, dinv2.reshape(2, N, 1),
                          conv_c_b.reshape(1, H), conv_o_b.reshape(1, H))
    return (mo.reshape(B, B), mc.reshape(B, B), mco.reshape(B, B),
            na1.reshape(-1))
